# baseline (speedup 1.0000x reference)
import functools

import jax
import jax.numpy as jnp
from jax import lax
from jax.experimental import pallas as pl
from jax.experimental.pallas import tpu as pltpu
from jax.experimental.pallas import tpu_sc as plsc

DIM = 2048
N_EXPERTS = 8
BLOCK_N = 512

NC = 2
NS = 16
L = 16


def _scores_block(x_ref, wt_ref, s_ref):
    scores = jnp.dot(x_ref[...], wt_ref[...], preferred_element_type=jnp.float32)
    st = scores.T
    m = jnp.max(st, axis=0, keepdims=True)
    e = jnp.exp(st - m)
    s_ref[...] = e / jnp.sum(e, axis=0, keepdims=True)


def _scores_tc(x, wt):
    n_tokens = x.shape[0]
    grid = (n_tokens // BLOCK_N,)
    return pl.pallas_call(
        _scores_block,
        grid=grid,
        in_specs=[
            pl.BlockSpec((BLOCK_N, DIM), lambda i: (i, 0)),
            pl.BlockSpec((DIM, N_EXPERTS), lambda i: (0, 0)),
        ],
        out_specs=pl.BlockSpec((N_EXPERTS, BLOCK_N), lambda i: (0, i)),
        out_shape=jax.ShapeDtypeStruct((N_EXPERTS, n_tokens), jnp.float32),
    )(x, wt)


def _route_sc(scores_t, n_tokens):
    nw = NC * NS
    b = n_tokens // nw
    mesh = plsc.VectorSubcoreMesh(core_axis_name="c", subcore_axis_name="s")

    @functools.partial(
        pl.kernel,
        mesh=mesh,
        out_type=[
            jax.ShapeDtypeStruct((2, n_tokens), jnp.float32),
            jax.ShapeDtypeStruct((2, n_tokens), jnp.int32),
        ],
        scratch_types=(
            [pltpu.VMEM((b,), jnp.float32) for _ in range(N_EXPERTS)]
            + [pltpu.VMEM((b,), jnp.float32) for _ in range(2)]
            + [pltpu.VMEM((b,), jnp.int32) for _ in range(2)]
        ),
    )
    def route(scores_hbm, w_hbm, i_hbm, *scratch):
        s_v = scratch[:N_EXPERTS]
        w1_v, w2_v = scratch[N_EXPERTS], scratch[N_EXPERTS + 1]
        i1_v, i2_v = scratch[N_EXPERTS + 2], scratch[N_EXPERTS + 3]
        wid = lax.axis_index("s") * NC + lax.axis_index("c")
        base = wid * b
        for e in range(N_EXPERTS):
            pltpu.sync_copy(scores_hbm.at[e, pl.ds(base, b)], s_v[e])

        def chunk(c, carry):
            off = c * L
            p = [s_v[e][pl.ds(off, L)] for e in range(N_EXPERTS)]
            w1 = p[0]
            i1 = jnp.zeros((L,), jnp.int32)
            w2 = jnp.full((L,), -1.0, jnp.float32)
            i2 = jnp.full((L,), N_EXPERTS, jnp.int32)
            for e in range(1, N_EXPERTS):
                ei = jnp.full((L,), e, jnp.int32)
                c1 = p[e] > w1
                c2 = p[e] > w2
                w2 = jnp.where(c1, w1, jnp.where(c2, p[e], w2))
                i2 = jnp.where(c1, i1, jnp.where(c2, ei, i2))
                w1 = jnp.where(c1, p[e], w1)
                i1 = jnp.where(c1, ei, i1)
            w1_v[pl.ds(off, L)] = w1
            w2_v[pl.ds(off, L)] = w2
            i1_v[pl.ds(off, L)] = i1
            i2_v[pl.ds(off, L)] = i2
            return carry

        lax.fori_loop(0, b // L, chunk, 0)
        pltpu.sync_copy(w1_v, w_hbm.at[0, pl.ds(base, b)])
        pltpu.sync_copy(w2_v, w_hbm.at[1, pl.ds(base, b)])
        pltpu.sync_copy(i1_v, i_hbm.at[0, pl.ds(base, b)])
        pltpu.sync_copy(i2_v, i_hbm.at[1, pl.ds(base, b)])

    return route(scores_t)


@jax.jit
def kernel(x, W):
    n_tokens = x.shape[0]
    wt = W.T
    scores_t = _scores_tc(x, wt)
    w_t, i_t = _route_sc(scores_t, n_tokens)
    return w_t.T, i_t.T


# speedup vs baseline: 1.2207x; 1.2207x over previous
import functools

import jax
import jax.numpy as jnp
from jax import lax
from jax.experimental import pallas as pl
from jax.experimental.pallas import tpu as pltpu
from jax.experimental.pallas import tpu_sc as plsc

DIM = 2048
N_EXPERTS = 8
BLOCK_N = 1024

NC = 2
NS = 16
L = 16


def _scores_block(x_ref, wt_ref, s_ref):
    scores = jnp.dot(x_ref[...], wt_ref[...], preferred_element_type=jnp.float32)
    st = scores.T
    m = jnp.max(st, axis=0, keepdims=True)
    e = jnp.exp(st - m)
    s_ref[...] = e / jnp.sum(e, axis=0, keepdims=True)


def _scores_tc(x, wt):
    n_tokens = x.shape[0]
    grid = (n_tokens // BLOCK_N,)
    return pl.pallas_call(
        _scores_block,
        grid=grid,
        in_specs=[
            pl.BlockSpec((BLOCK_N, DIM), lambda i: (i, 0)),
            pl.BlockSpec((DIM, N_EXPERTS), lambda i: (0, 0)),
        ],
        out_specs=pl.BlockSpec((N_EXPERTS, BLOCK_N), lambda i: (0, i)),
        out_shape=jax.ShapeDtypeStruct((N_EXPERTS, n_tokens), jnp.float32),
    )(x, wt)


def _route_sc(scores_t, n_tokens):
    nw = NC * NS
    b = n_tokens // nw
    mesh = plsc.VectorSubcoreMesh(core_axis_name="c", subcore_axis_name="s")

    @functools.partial(
        pl.kernel,
        mesh=mesh,
        out_type=[
            jax.ShapeDtypeStruct((2, n_tokens), jnp.float32),
            jax.ShapeDtypeStruct((2, n_tokens), jnp.int32),
        ],
        scratch_types=(
            [pltpu.VMEM((b,), jnp.float32) for _ in range(N_EXPERTS)]
            + [pltpu.VMEM((b,), jnp.float32) for _ in range(2)]
            + [pltpu.VMEM((b,), jnp.int32) for _ in range(2)]
            + [pltpu.SemaphoreType.DMA, pltpu.SemaphoreType.DMA]
        ),
    )
    def route(scores_hbm, w_hbm, i_hbm, *scratch):
        s_v = scratch[:N_EXPERTS]
        w1_v, w2_v = scratch[N_EXPERTS], scratch[N_EXPERTS + 1]
        i1_v, i2_v = scratch[N_EXPERTS + 2], scratch[N_EXPERTS + 3]
        in_sem, out_sem = scratch[N_EXPERTS + 4], scratch[N_EXPERTS + 5]
        wid = lax.axis_index("s") * NC + lax.axis_index("c")
        base = wid * b
        copies = [
            pltpu.make_async_copy(scores_hbm.at[e, pl.ds(base, b)], s_v[e], in_sem)
            for e in range(N_EXPERTS)
        ]
        for c in copies:
            c.start()
        for c in copies:
            c.wait()

        def chunk(c, carry):
            off = c * L
            p = [s_v[e][pl.ds(off, L)] for e in range(N_EXPERTS)]
            w1 = p[0]
            i1 = jnp.zeros((L,), jnp.int32)
            w2 = jnp.full((L,), -1.0, jnp.float32)
            i2 = jnp.full((L,), N_EXPERTS, jnp.int32)
            for e in range(1, N_EXPERTS):
                ei = jnp.full((L,), e, jnp.int32)
                c1 = p[e] > w1
                c2 = p[e] > w2
                w2 = jnp.where(c1, w1, jnp.where(c2, p[e], w2))
                i2 = jnp.where(c1, i1, jnp.where(c2, ei, i2))
                w1 = jnp.where(c1, p[e], w1)
                i1 = jnp.where(c1, ei, i1)
            w1_v[pl.ds(off, L)] = w1
            w2_v[pl.ds(off, L)] = w2
            i1_v[pl.ds(off, L)] = i1
            i2_v[pl.ds(off, L)] = i2
            return carry

        lax.fori_loop(0, b // L, chunk, 0)
        outs = [
            pltpu.make_async_copy(w1_v, w_hbm.at[0, pl.ds(base, b)], out_sem),
            pltpu.make_async_copy(w2_v, w_hbm.at[1, pl.ds(base, b)], out_sem),
            pltpu.make_async_copy(i1_v, i_hbm.at[0, pl.ds(base, b)], out_sem),
            pltpu.make_async_copy(i2_v, i_hbm.at[1, pl.ds(base, b)], out_sem),
        ]
        for c in outs:
            c.start()
        for c in outs:
            c.wait()

    return route(scores_t)


@jax.jit
def kernel(x, W):
    n_tokens = x.shape[0]
    wt = W.T
    scores_t = _scores_tc(x, wt)
    w_t, i_t = _route_sc(scores_t, n_tokens)
    return w_t.T, i_t.T
